# baseline (speedup 1.0000x reference)
import functools
import math

import jax
import jax.numpy as jnp
import numpy as np
from jax import lax
from jax.experimental import pallas as pl
from jax.experimental.pallas import tpu as pltpu
from jax.experimental.pallas import tpu_sc as plsc

VOCAB = 1000000
D = 64
DP = 128
L_SEQ = 200
B = 4096
N = B * L_SEQ
NC, NS, LANES = 2, 16, 16
NW = NC * NS
ROWS_PER_W = N // NW
G = 128
N_CHUNKS = ROWS_PER_W // G
GO = G * D // 128


def _pos_embed_np(d_model, max_len):
    pos = np.arange(0, max_len).reshape(-1, 1).astype(np.float32)
    div_term = np.exp(
        np.arange(0, d_model, 2).astype(np.float32) * -(math.log(10000.0) / d_model))
    pe = np.zeros((max_len, d_model), dtype=np.float32)
    pe[:, 0::2] = np.sin(pos * div_term)
    pe[:, 1::2] = np.cos(pos * div_term)
    return pe


_mesh = plsc.VectorSubcoreMesh(core_axis_name="c", subcore_axis_name="s",
                               num_cores=NC, num_subcores=NS)


@functools.partial(
    pl.kernel,
    out_type=jax.ShapeDtypeStruct((N * D // 128, 128), jnp.float32),
    mesh=_mesh,
    scratch_types=[
        pltpu.VMEM((2, G), jnp.int32),
        pltpu.VMEM((2, G), jnp.int32),
        pltpu.VMEM((2, G), jnp.int32),
        pltpu.VMEM((2, G), jnp.int32),
        pltpu.VMEM((2, G, DP), jnp.float32),
        pltpu.VMEM((2, 2 * G, DP), jnp.float32),
        pltpu.VMEM((2, GO, 128), jnp.float32),
        pltpu.SemaphoreType.DMA,
        pltpu.SemaphoreType.DMA,
        pltpu.SemaphoreType.DMA,
        pltpu.SemaphoreType.DMA,
        pltpu.SemaphoreType.DMA,
        pltpu.SemaphoreType.DMA,
    ],
)
def _embed_kernel(tok_hbm, tbla_hbm, tblb_hbm, seq_hbm, seg_hbm, out_hbm,
                  seqb, segb, aidxb, colb, tokb, addv, outb,
                  sem_t0, sem_t1, sem_a0, sem_a1, sem_o0, sem_o1):
    sem_t = (sem_t0, sem_t1)
    sem_a = (sem_a0, sem_a1)
    sem_o = (sem_o0, sem_o1)
    wid = lax.axis_index("s") * NC + lax.axis_index("c")
    gw = wid * N_CHUNKS
    iota16 = lax.iota(jnp.int32, 16)

    def prefetch_fire(c, p):
        gbase = gw + c
        rowbase = gbase * G
        lstart = rowbase % L_SEQ
        pltpu.sync_copy(seq_hbm.at[gbase], seqb.at[p])
        pltpu.sync_copy(seg_hbm.at[gbase], segb.at[p])

        for k in range(G // LANES):
            off = k * LANES
            seqv = seqb[p, pl.ds(off, LANES)]
            segv = segb[p, pl.ds(off, LANES)]
            aidxb[p, pl.ds(off, LANES)] = segv * G + (off + iota16)
            colb[p, pl.ds(off, LANES)] = jnp.where(
                seqv == 0, jnp.int32(D), jnp.int32(0))

        pltpu.async_copy(tok_hbm.at[seqb.at[p]], tokb.at[p], sem_t[p])
        pltpu.async_copy(tbla_hbm.at[pl.ds(lstart, G)],
                         addv.at[p, pl.ds(0, G)], sem_a[p])
        pltpu.async_copy(tblb_hbm.at[pl.ds(lstart, G)],
                         addv.at[p, pl.ds(G, G)], sem_a[p])

    def wait_add_out(c, p, drain_out):
        lstart = ((gw + c) * G) % L_SEQ
        pltpu.make_async_copy(
            tok_hbm.at[seqb.at[p]], tokb.at[p], sem_t[p]).wait()
        pltpu.make_async_copy(
            tbla_hbm.at[pl.ds(lstart, G)],
            addv.at[p, pl.ds(0, G)], sem_a[p]).wait()
        pltpu.make_async_copy(
            tblb_hbm.at[pl.ds(lstart, G)],
            addv.at[p, pl.ds(G, G)], sem_a[p]).wait()

        if drain_out:
            pltpu.make_async_copy(
                outb.at[p], out_hbm.at[pl.ds(0, GO)], sem_o[p]).wait()

        def add_slab(g, _):
            rsv = aidxb[p, pl.ds(g * LANES, LANES)]
            cov = colb[p, pl.ds(g * LANES, LANES)]
            for j in range(LANES):
                r = g * LANES + j
                r2 = g * 8 + j // 2
                rs = rsv[j]
                co = cov[j]
                for k in range(D // LANES):
                    outb[p, r2, pl.ds((j % 2) * D + k * LANES, LANES)] = (
                        tokb[p, r, pl.ds(k * LANES, LANES)]
                        + addv[p, rs, pl.ds(co + k * LANES, LANES)])
            return 0
        lax.fori_loop(0, G // LANES, add_slab, 0)

        sbase = (gw + c) * GO
        pltpu.async_copy(outb.at[p], out_hbm.at[pl.ds(sbase, GO)], sem_o[p])

    prefetch_fire(jnp.int32(0), 0)
    prefetch_fire(jnp.int32(1), 1)

    def pair_body(cc, _):
        for b in (0, 1):
            c = cc * 2 + b
            wait_add_out(c, b, True)

            @pl.when(c + 2 < N_CHUNKS)
            def _():
                prefetch_fire(c + 2, b)
        return 0

    for b in (0, 1):
        wait_add_out(jnp.int32(b), b, False)
        prefetch_fire(jnp.int32(2 + b), b)

    lax.fori_loop(1, N_CHUNKS // 2, pair_body, 0)

    pltpu.make_async_copy(outb.at[0], out_hbm.at[pl.ds(0, GO)], sem_o0).wait()
    pltpu.make_async_copy(outb.at[1], out_hbm.at[pl.ds(0, GO)], sem_o1).wait()


def kernel(sequence, segment_label, token_table, segment_table):
    seq = sequence.astype(jnp.int32).reshape(N // G, G)
    seg = segment_label.astype(jnp.int32).reshape(N // G, G)

    tokp = jnp.pad(token_table, ((0, 0), (0, DP - D)))

    pe = jnp.asarray(_pos_embed_np(D, L_SEQ))
    pe2 = jnp.concatenate([pe, pe], axis=0)
    seg1 = segment_table[1][None, :]
    tok0 = token_table[0][None, :]
    tbla = jnp.concatenate([pe2, pe2 - tok0], axis=1)
    tblb = jnp.concatenate([pe2 + seg1, pe2 + seg1 - tok0], axis=1)

    out = _embed_kernel(tokp, tbla, tblb, seq, seg)
    return out.reshape(B, L_SEQ, D)


# speedup vs baseline: 1.1623x; 1.1623x over previous
import functools
import math

import jax
import jax.numpy as jnp
import numpy as np
from jax import lax
from jax.experimental import pallas as pl
from jax.experimental.pallas import tpu as pltpu
from jax.experimental.pallas import tpu_sc as plsc

VOCAB = 1000000
D = 64
DP = 128
L_SEQ = 200
B = 4096
N = B * L_SEQ
NC, NS, LANES = 2, 16, 16
NW = NC * NS
ROWS_PER_W = N // NW
G = 128
N_CHUNKS = ROWS_PER_W // G
GO = G * D // 128


def _pos_embed_np(d_model, max_len):
    pos = np.arange(0, max_len).reshape(-1, 1).astype(np.float32)
    div_term = np.exp(
        np.arange(0, d_model, 2).astype(np.float32) * -(math.log(10000.0) / d_model))
    pe = np.zeros((max_len, d_model), dtype=np.float32)
    pe[:, 0::2] = np.sin(pos * div_term)
    pe[:, 1::2] = np.cos(pos * div_term)
    return pe


_mesh = plsc.VectorSubcoreMesh(core_axis_name="c", subcore_axis_name="s",
                               num_cores=NC, num_subcores=NS)


@functools.partial(
    pl.kernel,
    out_type=jax.ShapeDtypeStruct((N * D // 128, 128), jnp.float32),
    mesh=_mesh,
    scratch_types=[
        pltpu.VMEM((4, G), jnp.int32),
        pltpu.VMEM((4, G), jnp.int32),
        pltpu.VMEM((2, G), jnp.int32),
        pltpu.VMEM((2, G, DP), jnp.float32),
        pltpu.VMEM((2, G, DP), jnp.float32),
        pltpu.VMEM((2, GO, 128), jnp.float32),
        pltpu.SemaphoreType.DMA,
        pltpu.SemaphoreType.DMA,
        pltpu.SemaphoreType.DMA,
        pltpu.SemaphoreType.DMA,
        pltpu.SemaphoreType.DMA,
        pltpu.SemaphoreType.DMA,
        pltpu.SemaphoreType.DMA,
        pltpu.SemaphoreType.DMA,
        pltpu.SemaphoreType.DMA,
        pltpu.SemaphoreType.DMA,
    ],
)
def _embed_kernel(tok_hbm, add_hbm, seq_hbm, seg_hbm, out_hbm,
                  seqb, segb, aidxb, tokb, addb, outb,
                  sem_i0, sem_i1, sem_i2, sem_i3,
                  sem_t0, sem_t1, sem_a0, sem_a1, sem_o0, sem_o1):
    sem_i = (sem_i0, sem_i1, sem_i2, sem_i3)
    sem_t = (sem_t0, sem_t1)
    sem_a = (sem_a0, sem_a1)
    sem_o = (sem_o0, sem_o1)
    wid = lax.axis_index("s") * NC + lax.axis_index("c")
    gw = wid * N_CHUNKS
    iota16 = lax.iota(jnp.int32, 16)

    def fire_idx(c, q):
        gbase = gw + c
        pltpu.async_copy(seq_hbm.at[gbase], seqb.at[q], sem_i[q])
        pltpu.async_copy(seg_hbm.at[gbase], segb.at[q], sem_i[q])

    def fire_gathers(c, p, q):
        gbase = gw + c
        rowbase = gbase * G
        pltpu.make_async_copy(seq_hbm.at[gbase], seqb.at[q], sem_i[q]).wait()
        pltpu.make_async_copy(seg_hbm.at[gbase], segb.at[q], sem_i[q]).wait()

        for k in range(G // LANES):
            off = k * LANES
            rows = rowbase + off + iota16
            lmod = rows % L_SEQ
            seqv = seqb[q, pl.ds(off, LANES)]
            segv = segb[q, pl.ds(off, LANES)]
            aidx = lmod + segv * L_SEQ + jnp.where(
                seqv == 0, jnp.int32(2 * L_SEQ), jnp.int32(0))
            aidxb[p, pl.ds(off, LANES)] = aidx

        pltpu.async_copy(tok_hbm.at[seqb.at[q]], tokb.at[p], sem_t[p])
        pltpu.async_copy(add_hbm.at[aidxb.at[p]], addb.at[p], sem_a[p])

    def emit(c, p, q, drain_out):
        pltpu.make_async_copy(
            tok_hbm.at[seqb.at[q]], tokb.at[p], sem_t[p]).wait()
        pltpu.make_async_copy(
            add_hbm.at[aidxb.at[p]], addb.at[p], sem_a[p]).wait()

        if drain_out:
            pltpu.make_async_copy(
                outb.at[p], out_hbm.at[pl.ds(0, GO)], sem_o[p]).wait()

        def add_slab(s, _):
            for j in range(8):
                r = s * 8 + j
                r2 = s * 4 + j // 2
                for k in range(D // LANES):
                    outb[p, r2, pl.ds((j % 2) * D + k * LANES, LANES)] = (
                        tokb[p, r, pl.ds(k * LANES, LANES)]
                        + addb[p, r, pl.ds(k * LANES, LANES)])
            return 0
        lax.fori_loop(0, G // 8, add_slab, 0)

        sbase = (gw + c) * GO
        pltpu.async_copy(outb.at[p], out_hbm.at[pl.ds(sbase, GO)], sem_o[p])

    for c0 in range(4):
        fire_idx(jnp.int32(c0), c0)
    fire_gathers(jnp.int32(0), 0, 0)
    fire_gathers(jnp.int32(1), 1, 1)

    for c0 in range(4):
        emit(jnp.int32(c0), c0 % 2, c0 % 4, c0 >= 2)
        fire_idx(jnp.int32(c0 + 4), c0 % 4)
        fire_gathers(jnp.int32(c0 + 2), c0 % 2, (c0 + 2) % 4)

    def quad_body(cc, _):
        for b in range(4):
            c = cc * 4 + b
            emit(c, b % 2, b, True)
            fire_idx(c + 4, b)
            fire_gathers(c + 2, b % 2, (b + 2) % 4)
        return 0
    lax.fori_loop(1, N_CHUNKS // 4 - 1, quad_body, 0)

    for c0 in range(N_CHUNKS - 4, N_CHUNKS):
        b = c0 % 4
        emit(jnp.int32(c0), b % 2, b, True)

        if c0 + 2 < N_CHUNKS:
            fire_gathers(jnp.int32(c0 + 2), b % 2, (b + 2) % 4)

    pltpu.make_async_copy(outb.at[0], out_hbm.at[pl.ds(0, GO)], sem_o0).wait()
    pltpu.make_async_copy(outb.at[1], out_hbm.at[pl.ds(0, GO)], sem_o1).wait()


def kernel(sequence, segment_label, token_table, segment_table):
    seq = sequence.astype(jnp.int32).reshape(N // G, G)
    seg = segment_label.astype(jnp.int32).reshape(N // G, G)

    tokp = jnp.pad(token_table, ((0, 0), (0, DP - D)))

    pe = jnp.asarray(_pos_embed_np(D, L_SEQ))
    seg1 = segment_table[1][None, :]
    tok0 = token_table[0][None, :]
    add_tbl = jnp.concatenate(
        [pe, pe + seg1, pe - tok0, pe + seg1 - tok0], axis=0)
    add_tbl = jnp.pad(add_tbl, ((0, 0), (0, DP - D)))

    out = _embed_kernel(tokp, add_tbl, seq, seg)
    return out.reshape(B, L_SEQ, D)


# speedup vs baseline: 1.3698x; 1.1786x over previous
import functools
import math

import jax
import jax.numpy as jnp
import numpy as np
from jax import lax
from jax.experimental import pallas as pl
from jax.experimental.pallas import tpu as pltpu
from jax.experimental.pallas import tpu_sc as plsc

VOCAB = 1000000
D = 64
DP = 128
L_SEQ = 200
B = 4096
N = B * L_SEQ
NC, NS, LANES = 2, 16, 16
NW = NC * NS
ROWS_PER_W = N // NW
G = 128
N_CHUNKS = ROWS_PER_W // G
GO = G * D // 128


def _pos_embed_np(d_model, max_len):
    pos = np.arange(0, max_len).reshape(-1, 1).astype(np.float32)
    div_term = np.exp(
        np.arange(0, d_model, 2).astype(np.float32) * -(math.log(10000.0) / d_model))
    pe = np.zeros((max_len, d_model), dtype=np.float32)
    pe[:, 0::2] = np.sin(pos * div_term)
    pe[:, 1::2] = np.cos(pos * div_term)
    return pe


_mesh = plsc.VectorSubcoreMesh(core_axis_name="c", subcore_axis_name="s",
                               num_cores=NC, num_subcores=NS)


@functools.partial(
    pl.kernel,
    out_type=jax.ShapeDtypeStruct((N // 8, 8, D), jnp.float32),
    mesh=_mesh,
    scratch_types=[
        pltpu.VMEM((4, G), jnp.int32),
        pltpu.VMEM((4, G), jnp.int32),
        pltpu.VMEM((2, G), jnp.int32),
        pltpu.VMEM((2, G, DP), jnp.float32),
        pltpu.VMEM((2, G, DP), jnp.float32),
        pltpu.VMEM((2, G // 8, 8, D), jnp.float32),
        pltpu.SemaphoreType.DMA,
        pltpu.SemaphoreType.DMA,
        pltpu.SemaphoreType.DMA,
        pltpu.SemaphoreType.DMA,
        pltpu.SemaphoreType.DMA,
        pltpu.SemaphoreType.DMA,
        pltpu.SemaphoreType.DMA,
        pltpu.SemaphoreType.DMA,
        pltpu.SemaphoreType.DMA,
        pltpu.SemaphoreType.DMA,
    ],
)
def _embed_kernel(tok_hbm, add_hbm, seq_hbm, seg_hbm, out_hbm,
                  seqb, segb, aidxb, tokb, addb, outb,
                  sem_i0, sem_i1, sem_i2, sem_i3,
                  sem_t0, sem_t1, sem_a0, sem_a1, sem_o0, sem_o1):
    sem_i = (sem_i0, sem_i1, sem_i2, sem_i3)
    sem_t = (sem_t0, sem_t1)
    sem_a = (sem_a0, sem_a1)
    sem_o = (sem_o0, sem_o1)
    wid = lax.axis_index("s") * NC + lax.axis_index("c")
    gw = wid * N_CHUNKS
    iota16 = lax.iota(jnp.int32, 16)

    def fire_idx(c, q):
        gbase = gw + c
        pltpu.async_copy(seq_hbm.at[gbase], seqb.at[q], sem_i[q])
        pltpu.async_copy(seg_hbm.at[gbase], segb.at[q], sem_i[q])

    def fire_gathers(c, p, q):
        gbase = gw + c
        rowbase = gbase * G
        pltpu.make_async_copy(seq_hbm.at[gbase], seqb.at[q], sem_i[q]).wait()
        pltpu.make_async_copy(seg_hbm.at[gbase], segb.at[q], sem_i[q]).wait()

        for k in range(G // LANES):
            off = k * LANES
            rows = rowbase + off + iota16
            lmod = rows % L_SEQ
            seqv = seqb[q, pl.ds(off, LANES)]
            segv = segb[q, pl.ds(off, LANES)]
            aidx = lmod + segv * L_SEQ + jnp.where(
                seqv == 0, jnp.int32(2 * L_SEQ), jnp.int32(0))
            aidxb[p, pl.ds(off, LANES)] = aidx

        pltpu.async_copy(tok_hbm.at[seqb.at[q]], tokb.at[p], sem_t[p])
        pltpu.async_copy(add_hbm.at[aidxb.at[p]], addb.at[p], sem_a[p])

    def emit(c, p, q, drain_out):
        pltpu.make_async_copy(
            tok_hbm.at[seqb.at[q]], tokb.at[p], sem_t[p]).wait()
        pltpu.make_async_copy(
            add_hbm.at[aidxb.at[p]], addb.at[p], sem_a[p]).wait()

        if drain_out:
            pltpu.make_async_copy(
                outb.at[p], out_hbm.at[pl.ds(0, G // 8)], sem_o[p]).wait()

        def add_slab(s, _):
            for j in range(8):
                r = s * 8 + j
                for k in range(D // LANES):
                    sl = pl.ds(k * LANES, LANES)
                    outb[p, s, j, sl] = tokb[p, r, sl] + addb[p, r, sl]
            return 0
        lax.fori_loop(0, G // 8, add_slab, 0)

        sbase = (gw + c) * (G // 8)
        pltpu.async_copy(outb.at[p], out_hbm.at[pl.ds(sbase, G // 8)],
                         sem_o[p])

    for c0 in range(4):
        fire_idx(jnp.int32(c0), c0)
    fire_gathers(jnp.int32(0), 0, 0)
    fire_gathers(jnp.int32(1), 1, 1)

    for c0 in range(4):
        emit(jnp.int32(c0), c0 % 2, c0 % 4, c0 >= 2)
        fire_idx(jnp.int32(c0 + 4), c0 % 4)
        fire_gathers(jnp.int32(c0 + 2), c0 % 2, (c0 + 2) % 4)

    def quad_body(cc, _):
        for b in range(4):
            c = cc * 4 + b
            emit(c, b % 2, b, True)
            fire_idx(c + 4, b)
            fire_gathers(c + 2, b % 2, (b + 2) % 4)
        return 0
    lax.fori_loop(1, N_CHUNKS // 4 - 1, quad_body, 0)

    for c0 in range(N_CHUNKS - 4, N_CHUNKS):
        b = c0 % 4
        emit(jnp.int32(c0), b % 2, b, True)

        if c0 + 2 < N_CHUNKS:
            fire_gathers(jnp.int32(c0 + 2), b % 2, (b + 2) % 4)

    pltpu.make_async_copy(
        outb.at[0], out_hbm.at[pl.ds(0, G // 8)], sem_o0).wait()
    pltpu.make_async_copy(
        outb.at[1], out_hbm.at[pl.ds(0, G // 8)], sem_o1).wait()


def kernel(sequence, segment_label, token_table, segment_table):
    seq = sequence.astype(jnp.int32).reshape(N // G, G)
    seg = segment_label.astype(jnp.int32).reshape(N // G, G)

    tokp = jnp.pad(token_table, ((0, 0), (0, DP - D)))

    pe = jnp.asarray(_pos_embed_np(D, L_SEQ))
    seg1 = segment_table[1][None, :]
    tok0 = token_table[0][None, :]
    add_tbl = jnp.concatenate(
        [pe, pe + seg1, pe - tok0, pe + seg1 - tok0], axis=0)
    add_tbl = jnp.pad(add_tbl, ((0, 0), (0, DP - D)))

    out = _embed_kernel(tokp, add_tbl, seq, seg)
    return out.reshape(B, L_SEQ, D)
